# Initial kernel scaffold; baseline (speedup 1.0000x reference)
import jax
import jax.numpy as jnp
from jax.experimental import pallas as pl


def kernel(x, edge_index, W1, b1, W2, b2):
    raise NotImplementedError("write your pallas kernel here")


# speedup vs baseline: 16.6690x; 16.6690x over previous
import functools

import jax
import jax.numpy as jnp
from jax import lax
from jax.experimental import pallas as pl
from jax.experimental.pallas import tpu as pltpu
from jax.experimental.pallas import tpu_sc as plsc

NC = 2
NS = 16
NW = NC * NS
LANES = 16
CHUNK = 128


def _make_hist_kernel(NT, NCHUNK):
  mesh = plsc.VectorSubcoreMesh(core_axis_name="c", subcore_axis_name="s")
  rpt = NT // NS

  @functools.partial(
      pl.kernel,
      out_type=jax.ShapeDtypeStruct((NC, NT, LANES), jnp.float32),
      mesh=mesh,
      scratch_types=[
          pltpu.VMEM((NCHUNK, CHUNK), jnp.int32),
          pltpu.VMEM((CHUNK, LANES), jnp.float32),
          pltpu.VMEM_SHARED((NT, LANES), jnp.float32),
      ],
  )
  def hist_kernel(dst_hbm, ones_hbm, zeros_hbm, out_hbm, idx_v, ones_v,
                  hist_sh):
    c = lax.axis_index("c")
    s = lax.axis_index("s")
    wid = c * NS + s
    base = s * rpt
    pltpu.sync_copy(zeros_hbm.at[pl.ds(base, rpt)], hist_sh.at[pl.ds(base, rpt)])
    pltpu.sync_copy(ones_hbm, ones_v)
    pltpu.sync_copy(dst_hbm.at[wid], idx_v)
    plsc.subcore_barrier()

    def body(j, carry):
      pltpu.sync_copy(ones_v, hist_sh.at[idx_v.at[j]], add=True)
      return carry

    lax.fori_loop(0, NCHUNK, body, 0)
    plsc.subcore_barrier()
    pltpu.sync_copy(hist_sh.at[pl.ds(base, rpt)],
                    out_hbm.at[c, pl.ds(base, rpt)])

  return hist_kernel


def _make_agg_kernel(NT, NCHUNK, D):
  mesh = plsc.VectorSubcoreMesh(core_axis_name="c", subcore_axis_name="s")
  rpt = NT // NS

  @functools.partial(
      pl.kernel,
      out_type=jax.ShapeDtypeStruct((NC, NT, D), jnp.float32),
      mesh=mesh,
      compiler_params=pltpu.CompilerParams(use_tc_tiling_on_sc=False),
      scratch_types=[
          pltpu.VMEM((NCHUNK, CHUNK), jnp.int32),
          pltpu.VMEM((NCHUNK, CHUNK), jnp.int32),
          pltpu.VMEM((CHUNK, D), jnp.float32),
          pltpu.VMEM_SHARED((NT, D), jnp.float32),
          pltpu.SemaphoreType.DMA,
      ],
  )
  def agg_kernel(hs_hbm, src_hbm, dst_hbm, zeros_hbm, out_hbm,
                 sidx, didx, rows, agg_sh, sem):
    c = lax.axis_index("c")
    s = lax.axis_index("s")
    wid = c * NS + s
    base = s * rpt
    pltpu.sync_copy(zeros_hbm.at[pl.ds(base, rpt)], agg_sh.at[pl.ds(base, rpt)])
    pltpu.sync_copy(src_hbm.at[wid], sidx)
    pltpu.sync_copy(dst_hbm.at[wid], didx)
    plsc.subcore_barrier()

    def body(j, carry):
      pltpu.async_copy(hs_hbm.at[sidx.at[j]], rows, sem).wait()
      pltpu.sync_copy(rows, agg_sh.at[didx.at[j]], add=True)
      return carry

    lax.fori_loop(0, NCHUNK, body, 0)
    plsc.subcore_barrier()
    pltpu.sync_copy(agg_sh.at[pl.ds(base, rpt)],
                    out_hbm.at[c, pl.ds(base, rpt)])

  return agg_kernel


def _dinv_from_hist(hist_ref):
  deg = hist_ref[0, :, 0:1] + hist_ref[1, :, 0:1] + 1.0
  return lax.rsqrt(deg)


def _matmul_scale_body(x_ref, w_ref, hist_ref, out_ref):
  dinv = _dinv_from_hist(hist_ref)
  h = jnp.dot(x_ref[...], w_ref[...], preferred_element_type=jnp.float32)
  out_ref[...] = h * dinv


def _mid_layer_body(agg_ref, hs_ref, hist_ref, w_ref, b_ref, out_ref):
  dinv = _dinv_from_hist(hist_ref)
  t = (agg_ref[0] + agg_ref[1] + hs_ref[...]) * dinv + b_ref[...]
  h = jnp.maximum(t, 0.0)
  out_ref[...] = jnp.dot(h, w_ref[...], preferred_element_type=jnp.float32) * dinv


def _final_body(agg_ref, hs_ref, hist_ref, b_ref, out_ref):
  dinv = _dinv_from_hist(hist_ref)
  out_ref[...] = (agg_ref[0] + agg_ref[1] + hs_ref[...]) * dinv + b_ref[...]


def kernel(x, edge_index, W1, b1, W2, b2):
  N, DIN = x.shape
  DH = W1.shape[1]
  DOUT = W2.shape[1]
  E = edge_index.shape[1]

  x = x.astype(jnp.float32)
  src = edge_index[0].astype(jnp.int32)
  dst = edge_index[1].astype(jnp.int32)

  NT = -(-(N + 1) // (NS * 8)) * (NS * 8)
  NCHUNK = -(-E // (NW * CHUNK))
  E_pad = NW * NCHUNK * CHUNK
  src_p = jnp.concatenate([src, jnp.zeros((E_pad - E,), jnp.int32)])
  dst_p = jnp.concatenate([dst, jnp.full((E_pad - E,), N, jnp.int32)])
  src3 = src_p.reshape(NW, NCHUNK, CHUNK)
  dst3 = dst_p.reshape(NW, NCHUNK, CHUNK)

  ones_l = jnp.ones((CHUNK, LANES), jnp.float32)
  zeros_l = jnp.zeros((NT, LANES), jnp.float32)
  zeros_h = jnp.zeros((NT, DH), jnp.float32)
  zeros_o = jnp.zeros((NT, DOUT), jnp.float32)

  hist = _make_hist_kernel(NT, NCHUNK)(dst3, ones_l, zeros_l)

  bm = 1024
  grid = (pl.cdiv(N, bm),)
  hist_spec = pl.BlockSpec((NC, bm, LANES), lambda m: (0, m, 0))

  hs1 = pl.pallas_call(
      _matmul_scale_body,
      grid=grid,
      in_specs=[
          pl.BlockSpec((bm, DIN), lambda m: (m, 0)),
          pl.BlockSpec((DIN, DH), lambda m: (0, 0)),
          hist_spec,
      ],
      out_specs=pl.BlockSpec((bm, DH), lambda m: (m, 0)),
      out_shape=jax.ShapeDtypeStruct((N, DH), jnp.float32),
  )(x, W1, hist)

  agg1 = _make_agg_kernel(NT, NCHUNK, DH)(hs1, src3, dst3, zeros_h)

  hs2 = pl.pallas_call(
      _mid_layer_body,
      grid=grid,
      in_specs=[
          pl.BlockSpec((NC, bm, DH), lambda m: (0, m, 0)),
          pl.BlockSpec((bm, DH), lambda m: (m, 0)),
          hist_spec,
          pl.BlockSpec((DH, DOUT), lambda m: (0, 0)),
          pl.BlockSpec((1, DH), lambda m: (0, 0)),
      ],
      out_specs=pl.BlockSpec((bm, DOUT), lambda m: (m, 0)),
      out_shape=jax.ShapeDtypeStruct((N, DOUT), jnp.float32),
  )(agg1, hs1, hist, W2, b1.reshape(1, DH))

  agg2 = _make_agg_kernel(NT, NCHUNK, DOUT)(hs2, src3, dst3, zeros_o)

  out = pl.pallas_call(
      _final_body,
      grid=grid,
      in_specs=[
          pl.BlockSpec((NC, bm, DOUT), lambda m: (0, m, 0)),
          pl.BlockSpec((bm, DOUT), lambda m: (m, 0)),
          hist_spec,
          pl.BlockSpec((1, DOUT), lambda m: (0, 0)),
      ],
      out_specs=pl.BlockSpec((bm, DOUT), lambda m: (m, 0)),
      out_shape=jax.ShapeDtypeStruct((N, DOUT), jnp.float32),
  )(agg2, hs2, hist, b2.reshape(1, DOUT))

  return out
